# Initial kernel scaffold; baseline (speedup 1.0000x reference)
import jax
import jax.numpy as jnp
from jax.experimental import pallas as pl


def kernel(inputs, state, sup_rows, sup_cols, sup_vals, W_gate, b_gate, W_cand, b_cand):
    raise NotImplementedError("write your pallas kernel here")


# speedup vs baseline: 1.1810x; 1.1810x over previous
import functools

import jax
import jax.numpy as jnp
from jax.experimental import pallas as pl

N = 10000
DEG = 16
B = 8
IN_DIM = 2
U = 64
K = 2
NS = 2
ISZ = IN_DIM + U
M = NS * K + 1
RB = 2000


def _spmm(r, c, v, x):
    return jax.ops.segment_sum(v[:, None] * jnp.take(x, c, axis=0), r, num_segments=N)


def _diffuse(x0, sup_rows, sup_cols, sup_vals):
    xs = [x0]
    for s in range(NS):
        r, c, v = sup_rows[s], sup_cols[s], sup_vals[s]
        x1 = _spmm(r, c, v, x0)
        xs.append(x1)
        xkm1, xkm2 = x1, x0
        for _ in range(2, K + 1):
            x2 = 2.0 * _spmm(r, c, v, xkm1) - xkm2
            xs.append(x2)
            xkm1, xkm2 = x2, xkm1
    return xs


def _gate_body(x0_ref, x1_ref, x2_ref, x3_ref, x4_ref, w_ref, bias_ref, st_ref,
               rs_ref, u_ref):
    acc = jnp.broadcast_to(bias_ref[...], (RB, 2 * U)).astype(jnp.float32)
    for m, xr in enumerate((x0_ref, x1_ref, x2_ref, x3_ref, x4_ref)):
        acc = acc + jnp.dot(xr[...], w_ref[m], preferred_element_type=jnp.float32)
    val = jax.nn.sigmoid(acc)
    r = val[:, :U]
    u = val[:, U:]
    rs_ref[...] = r * st_ref[...]
    u_ref[...] = u


def _cand_body(x0_ref, x1_ref, x2_ref, x3_ref, x4_ref, w_ref, bias_ref, u_ref,
               st_ref, ns_ref):
    acc = jnp.broadcast_to(bias_ref[...], (RB, U)).astype(jnp.float32)
    for m, xr in enumerate((x0_ref, x1_ref, x2_ref, x3_ref, x4_ref)):
        acc = acc + jnp.dot(xr[...], w_ref[m], preferred_element_type=jnp.float32)
    c = jnp.tanh(acc)
    u = u_ref[...]
    ns_ref[...] = u * st_ref[...] + (1.0 - u) * c


def _row_spec():
    return pl.BlockSpec((RB, ISZ), lambda i: (i, 0))


def _vec_spec(width):
    return pl.BlockSpec((RB, width), lambda i: (i, 0))


def _full_spec(shape):
    return pl.BlockSpec(shape, lambda i: (0,) * len(shape))


def _gate_call(xs_nb, w_g, b_g, st_nb):
    grid = (N * B // RB,)
    return pl.pallas_call(
        _gate_body,
        grid=grid,
        in_specs=[_row_spec()] * 5 + [_full_spec((M, ISZ, 2 * U)), _full_spec((1, 2 * U)), _vec_spec(U)],
        out_specs=[_vec_spec(U), _vec_spec(U)],
        out_shape=[jax.ShapeDtypeStruct((N * B, U), jnp.float32),
                   jax.ShapeDtypeStruct((N * B, U), jnp.float32)],
    )(*xs_nb, w_g, b_g, st_nb)


def _cand_call(xs_nb, w_c, b_c, u_nb, st_nb):
    grid = (N * B // RB,)
    return pl.pallas_call(
        _cand_body,
        grid=grid,
        in_specs=[_row_spec()] * 5 + [_full_spec((M, ISZ, U)), _full_spec((1, U)), _vec_spec(U), _vec_spec(U)],
        out_specs=_vec_spec(U),
        out_shape=jax.ShapeDtypeStruct((N * B, U), jnp.float32),
    )(*xs_nb, w_c, b_c, u_nb, st_nb)


def kernel(inputs, state, sup_rows, sup_cols, sup_vals, W_gate, b_gate, W_cand, b_cand):
    inp_nbf = jnp.transpose(inputs.reshape(B, N, IN_DIM), (1, 0, 2))
    st_nbf = jnp.transpose(state.reshape(B, N, U), (1, 0, 2))
    x0 = jnp.concatenate([inp_nbf, st_nbf], axis=2).reshape(N, B * ISZ)

    wg = W_gate.reshape(ISZ, M, 2 * U).transpose(1, 0, 2)
    wc = W_cand.reshape(ISZ, M, U).transpose(1, 0, 2)

    xs = _diffuse(x0, sup_rows, sup_cols, sup_vals)
    xs_nb = [x.reshape(N * B, ISZ) for x in xs]
    st_nb = st_nbf.reshape(N * B, U)

    rs_nb, u_nb = _gate_call(xs_nb, wg, b_gate.reshape(1, 2 * U), st_nb)

    x0c = jnp.concatenate([inp_nbf.reshape(N * B, IN_DIM), rs_nb], axis=1).reshape(N, B * ISZ)
    xs2 = _diffuse(x0c, sup_rows, sup_cols, sup_vals)
    xs2_nb = [x.reshape(N * B, ISZ) for x in xs2]

    ns_nb = _cand_call(xs2_nb, wc, b_cand.reshape(1, U), u_nb, st_nb)
    ns = jnp.transpose(ns_nb.reshape(N, B, U), (1, 0, 2)).reshape(B, N * U)
    return (ns, ns)


# speedup vs baseline: 2.1535x; 1.8234x over previous
import functools

import jax
import jax.numpy as jnp
from jax import lax
from jax.experimental import pallas as pl
from jax.experimental.pallas import tpu as pltpu
from jax.experimental.pallas import tpu_sc as plsc

N = 10000
DEG = 16
NNZ = N * DEG
B = 8
IN_DIM = 2
U = 64
K = 2
NS = 2
ISZ = IN_DIM + U
M = NS * K + 1
RB = 2000


def _spmm(r, c, v, x):
    return jax.ops.segment_sum(v[:, None] * jnp.take(x, c, axis=0), r, num_segments=N)


LANES = 16
FW = B * ISZ
FCH = FW // LANES
_NW = 32
_RPW_HI = 313
_RPW_LO = 312
_ECH = _RPW_HI * DEG


def _a1_body(minus, *refs):
    if minus:
        (x_hbm, cols_hbm, vals_hbm, xp_hbm, out_hbm,
         colbuf, valbuf, gbuf, obuf, pbuf, sem, psem) = refs
    else:
        (x_hbm, cols_hbm, vals_hbm, out_hbm,
         colbuf, valbuf, gbuf, obuf, sem) = refs
    c = lax.axis_index("c")
    s = lax.axis_index("s")
    wid = s * 2 + c
    hi = wid < 16
    base_row = jnp.where(hi, _RPW_HI * wid, _RPW_HI * 16 + _RPW_LO * (wid - 16))
    nr = jnp.where(hi, _RPW_HI, _RPW_LO)
    ebase = base_row * DEG
    pltpu.sync_copy(cols_hbm.at[pl.ds(ebase, _ECH)], colbuf)
    pltpu.sync_copy(vals_hbm.at[pl.ds(ebase, _ECH)], valbuf)

    def row_fn(i, carry):
        row = base_row + i
        idx = colbuf.at[pl.ds(i * DEG, DEG)]
        cp = pltpu.async_copy(x_hbm.at[idx], gbuf, sem)
        if minus:
            pp = pltpu.async_copy(xp_hbm.at[row], pbuf, psem)
        cp.wait()
        if minus:
            pp.wait()
        vvec = valbuf[pl.ds(i * DEG, LANES)]
        vbs = [jnp.broadcast_to(vvec[j], (LANES,)) for j in range(DEG)]

        def fc_fn(t, carry2):
            for u in range(3):
                sl = pl.ds((t * 3 + u) * LANES, LANES)
                acc = vbs[0] * gbuf[0, sl]
                for j in range(1, DEG):
                    acc = acc + vbs[j] * gbuf[j, sl]
                if minus:
                    acc = 2.0 * acc - pbuf[sl]
                obuf[sl] = acc
            return carry2

        lax.fori_loop(0, FCH // 3, fc_fn, 0)
        pltpu.sync_copy(obuf, out_hbm.at[row])
        return carry

    lax.fori_loop(0, nr, row_fn, 0)


_SLAB_OFF = (0, 144, 272, 400)
_SLAB_W = (144, 128, 128, 128)
_SLAB_MAX = 144
_SRC_PER_TILE = N // 16
_XCH = 25
_NCHUNK = _SRC_PER_TILE // _XCH


def _a2_slab(off, w, minus, s, x_hbm, xp_hbm, out_hbm, shared, colbuf, valbuf,
             xcbuf, sbuf, wbuf, pbuf):
    nq = w // LANES
    tbase = s * _SRC_PER_TILE

    def zb_fn(i, carry):
        zv = jnp.zeros((LANES,), jnp.float32)
        for q in range(_SLAB_MAX // LANES):
            xcbuf[i, pl.ds(q * LANES, LANES)] = zv
        return carry

    lax.fori_loop(0, _XCH, zb_fn, 0)

    def zc_fn(z, carry):
        pltpu.sync_copy(xcbuf, shared.at[pl.ds(tbase + z * _XCH, _XCH)])
        return carry

    lax.fori_loop(0, _NCHUNK, zc_fn, 0)
    plsc.subcore_barrier()

    def chunk_fn(k, carry):
        r0 = tbase + k * _XCH
        pltpu.sync_copy(x_hbm.at[pl.ds(r0, _XCH), pl.ds(off, w)],
                        xcbuf.at[:, pl.ds(0, w)])

        def row_fn(lr, carry2):
            er = k * _XCH + lr
            vv = valbuf[er]
            for j in range(DEG):
                vb = jnp.broadcast_to(vv[j], (LANES,))
                for q in range(nq):
                    sl = pl.ds(q * LANES, LANES)
                    sbuf[j, sl] = vb * xcbuf[lr, sl]
            idx = colbuf.at[er]
            pltpu.sync_copy(sbuf, shared.at[idx], add=True)
            return carry2

        lax.fori_loop(0, _XCH, row_fn, 0)
        return carry

    lax.fori_loop(0, _NCHUNK, chunk_fn, 0)
    plsc.subcore_barrier()

    def wchunk_fn(z, carry):
        rows = pl.ds(tbase + z * _XCH, _XCH)
        if minus:
            pltpu.sync_copy(shared.at[rows, pl.ds(0, w)], wbuf.at[:, pl.ds(0, w)])
            pltpu.sync_copy(xp_hbm.at[rows, pl.ds(off, w)], pbuf.at[:, pl.ds(0, w)])

            def wb_fn(i, carry3):
                for q in range(nq):
                    sl = pl.ds(q * LANES, LANES)
                    wbuf[i, sl] = 2.0 * wbuf[i, sl] - pbuf[i, sl]
                return carry3

            lax.fori_loop(0, _XCH, wb_fn, 0)
            pltpu.sync_copy(wbuf.at[:, pl.ds(0, w)], out_hbm.at[rows, pl.ds(off, w)])
        else:
            pltpu.sync_copy(shared.at[rows, pl.ds(0, w)], out_hbm.at[rows, pl.ds(off, w)])
        return carry

    lax.fori_loop(0, _NCHUNK, wchunk_fn, 0)


def _a2_body(minus, *refs):
    if minus:
        (x_hbm, cols_hbm, vals_hbm, xp_hbm, out_hbm,
         shared, colbuf, valbuf, xcbuf, sbuf, wbuf, pbuf) = refs
    else:
        (x_hbm, cols_hbm, vals_hbm, out_hbm,
         shared, colbuf, valbuf, xcbuf, sbuf, wbuf) = refs
        xp_hbm = None
        pbuf = None
    c = lax.axis_index("c")
    s = lax.axis_index("s")
    tbase = s * _SRC_PER_TILE
    pltpu.sync_copy(cols_hbm.at[pl.ds(tbase, _SRC_PER_TILE)], colbuf)
    pltpu.sync_copy(vals_hbm.at[pl.ds(tbase, _SRC_PER_TILE)], valbuf)
    for s_loc in range(2):
        for cc in range(2):
            sid = cc * 2 + s_loc

            @pl.when(c == cc)
            def _():
                _a2_slab(_SLAB_OFF[sid], _SLAB_W[sid], minus, s, x_hbm, xp_hbm,
                         out_hbm, shared, colbuf, valbuf, xcbuf, sbuf,
                         wbuf, pbuf)


def _sc_spmm_a2(x, cols2d, vals2d, xprev=None):
    minus = xprev is not None
    scratch = [
        pltpu.VMEM_SHARED((N, _SLAB_MAX), jnp.float32),
        pltpu.VMEM((_SRC_PER_TILE, DEG), jnp.int32),
        pltpu.VMEM((_SRC_PER_TILE, DEG), jnp.float32),
        pltpu.VMEM((_XCH, _SLAB_MAX), jnp.float32),
        pltpu.VMEM((DEG, _SLAB_MAX), jnp.float32),
        pltpu.VMEM((_XCH, _SLAB_MAX), jnp.float32),
    ]
    if minus:
        scratch.append(pltpu.VMEM((_XCH, _SLAB_MAX), jnp.float32))
    kfn = pl.kernel(
        functools.partial(_a2_body, minus),
        mesh=plsc.VectorSubcoreMesh(core_axis_name="c", subcore_axis_name="s"),
        out_type=jax.ShapeDtypeStruct((N, FW), jnp.float32),
        scratch_types=scratch,
        compiler_params=pltpu.CompilerParams(use_tc_tiling_on_sc=False),
    )
    args = (x, cols2d, vals2d) + ((xprev,) if minus else ())
    return kfn(*args)


def _sc_spmm_a1(x, cols, vals, xprev=None):
    minus = xprev is not None
    scratch = [
        pltpu.VMEM((_ECH,), jnp.int32),
        pltpu.VMEM((_ECH,), jnp.float32),
        pltpu.VMEM((DEG, FW), jnp.float32),
        pltpu.VMEM((FW,), jnp.float32),
    ]
    if minus:
        scratch.append(pltpu.VMEM((FW,), jnp.float32))
        scratch.append(pltpu.SemaphoreType.DMA)
        scratch.append(pltpu.SemaphoreType.DMA)
    else:
        scratch.append(pltpu.SemaphoreType.DMA)
    kfn = pl.kernel(
        functools.partial(_a1_body, minus),
        mesh=plsc.VectorSubcoreMesh(core_axis_name="c", subcore_axis_name="s"),
        out_type=jax.ShapeDtypeStruct((N, FW), jnp.float32),
        scratch_types=scratch,
        compiler_params=pltpu.CompilerParams(use_tc_tiling_on_sc=False),
    )
    args = (x, cols, vals) + ((xprev,) if minus else ())
    return kfn(*args)


def _diffuse(x0, cols0p, vals0p, dst2d, vals2d):
    y1 = _sc_spmm_a1(x0, cols0p, vals0p)
    y2 = _sc_spmm_a1(y1, cols0p, vals0p, xprev=x0)
    y3 = _sc_spmm_a2(x0, dst2d, vals2d)
    y4 = _sc_spmm_a2(y3, dst2d, vals2d, xprev=x0)
    return [x0, y1, y2, y3, y4]


def _gate_body(x0_ref, x1_ref, x2_ref, x3_ref, x4_ref, w_ref, bias_ref, st_ref,
               rs_ref, u_ref):
    acc = jnp.broadcast_to(bias_ref[...], (RB, 2 * U)).astype(jnp.float32)
    for m, xr in enumerate((x0_ref, x1_ref, x2_ref, x3_ref, x4_ref)):
        acc = acc + jnp.dot(xr[...], w_ref[m], preferred_element_type=jnp.float32)
    val = jax.nn.sigmoid(acc)
    r = val[:, :U]
    u = val[:, U:]
    rs_ref[...] = r * st_ref[...]
    u_ref[...] = u


def _cand_body(x0_ref, x1_ref, x2_ref, x3_ref, x4_ref, w_ref, bias_ref, u_ref,
               st_ref, ns_ref):
    acc = jnp.broadcast_to(bias_ref[...], (RB, U)).astype(jnp.float32)
    for m, xr in enumerate((x0_ref, x1_ref, x2_ref, x3_ref, x4_ref)):
        acc = acc + jnp.dot(xr[...], w_ref[m], preferred_element_type=jnp.float32)
    c = jnp.tanh(acc)
    u = u_ref[...]
    ns_ref[...] = u * st_ref[...] + (1.0 - u) * c


def _row_spec():
    return pl.BlockSpec((RB, ISZ), lambda i: (i, 0))


def _vec_spec(width):
    return pl.BlockSpec((RB, width), lambda i: (i, 0))


def _full_spec(shape):
    return pl.BlockSpec(shape, lambda i: (0,) * len(shape))


def _gate_call(xs_nb, w_g, b_g, st_nb):
    grid = (N * B // RB,)
    return pl.pallas_call(
        _gate_body,
        grid=grid,
        in_specs=[_row_spec()] * 5 + [_full_spec((M, ISZ, 2 * U)), _full_spec((1, 2 * U)), _vec_spec(U)],
        out_specs=[_vec_spec(U), _vec_spec(U)],
        out_shape=[jax.ShapeDtypeStruct((N * B, U), jnp.float32),
                   jax.ShapeDtypeStruct((N * B, U), jnp.float32)],
    )(*xs_nb, w_g, b_g, st_nb)


def _cand_call(xs_nb, w_c, b_c, u_nb, st_nb):
    grid = (N * B // RB,)
    return pl.pallas_call(
        _cand_body,
        grid=grid,
        in_specs=[_row_spec()] * 5 + [_full_spec((M, ISZ, U)), _full_spec((1, U)), _vec_spec(U), _vec_spec(U)],
        out_specs=_vec_spec(U),
        out_shape=jax.ShapeDtypeStruct((N * B, U), jnp.float32),
    )(*xs_nb, w_c, b_c, u_nb, st_nb)


def kernel(inputs, state, sup_rows, sup_cols, sup_vals, W_gate, b_gate, W_cand, b_cand):
    inp_nbf = jnp.transpose(inputs.reshape(B, N, IN_DIM), (1, 0, 2))
    st_nbf = jnp.transpose(state.reshape(B, N, U), (1, 0, 2))
    x0 = jnp.concatenate([inp_nbf, st_nbf], axis=2).reshape(N, B * ISZ)

    wg = W_gate.reshape(ISZ, M, 2 * U).transpose(1, 0, 2)
    wc = W_cand.reshape(ISZ, M, U).transpose(1, 0, 2)

    cols0p = jnp.pad(sup_cols[0], (0, 16))
    vals0p = jnp.pad(sup_vals[0], (0, 16))
    dst2d = sup_rows[1].reshape(NNZ // DEG, DEG)
    vals2d = sup_vals[1].reshape(NNZ // DEG, DEG)

    xs = _diffuse(x0, cols0p, vals0p, dst2d, vals2d)
    xs_nb = [x.reshape(N * B, ISZ) for x in xs]
    st_nb = st_nbf.reshape(N * B, U)

    rs_nb, u_nb = _gate_call(xs_nb, wg, b_gate.reshape(1, 2 * U), st_nb)

    x0c = jnp.concatenate([inp_nbf.reshape(N * B, IN_DIM), rs_nb], axis=1).reshape(N, B * ISZ)
    xs2 = _diffuse(x0c, cols0p, vals0p, dst2d, vals2d)
    xs2_nb = [x.reshape(N * B, ISZ) for x in xs2]

    ns_nb = _cand_call(xs2_nb, wc, b_cand.reshape(1, U), u_nb, st_nb)
    ns = jnp.transpose(ns_nb.reshape(N, B, U), (1, 0, 2)).reshape(B, N * U)
    return (ns, ns)


# speedup vs baseline: 4.1543x; 1.9291x over previous
import functools

import jax
import jax.numpy as jnp
from jax import lax
from jax.experimental import pallas as pl
from jax.experimental.pallas import tpu as pltpu
from jax.experimental.pallas import tpu_sc as plsc

N = 10000
DEG = 16
NNZ = N * DEG
B = 8
IN_DIM = 2
U = 64
K = 2
NS = 2
ISZ = IN_DIM + U
M = NS * K + 1
RB = 2000


def _spmm(r, c, v, x):
    return jax.ops.segment_sum(v[:, None] * jnp.take(x, c, axis=0), r, num_segments=N)


LANES = 16
FW = B * ISZ
FCH = FW // LANES
_NW = 32
_NP = 10112
_RPW = _NP // _NW
_ECH = _RPW * DEG
_EPAD = _NP * DEG


def _a1_body(minus, *refs):
    if minus:
        (x_hbm, cols_hbm, vals_hbm, xp_hbm, out_hbm, colbuf, valbuf,
         gbufA, gbufB, obufA, obufB, pbufA, pbufB,
         gsemA, gsemB, osemA, osemB, psemA, psemB) = refs
    else:
        (x_hbm, cols_hbm, vals_hbm, out_hbm, colbuf, valbuf,
         gbufA, gbufB, obufA, obufB,
         gsemA, gsemB, osemA, osemB) = refs
        xp_hbm = pbufA = pbufB = psemA = psemB = None
    c = lax.axis_index("c")
    s = lax.axis_index("s")
    wid = s * 2 + c
    base_row = wid * _RPW
    ebase = base_row * DEG
    pltpu.sync_copy(cols_hbm.at[pl.ds(ebase, _ECH)], colbuf)
    pltpu.sync_copy(vals_hbm.at[pl.ds(ebase, _ECH)], valbuf)

    def start_g(i, gb, gs):
        pltpu.async_copy(x_hbm.at[colbuf.at[pl.ds(i * DEG, DEG)]], gb, gs)

    def start_p(i, pb, ps):
        rowc = jnp.minimum(base_row + i, N - 1)
        pltpu.async_copy(xp_hbm.at[rowc], pb, ps)

    def wait_g(gb, gs):
        pltpu.make_async_copy(x_hbm.at[colbuf.at[pl.ds(0, DEG)]], gb, gs).wait()

    def wait_p(pb, ps):
        pltpu.make_async_copy(xp_hbm.at[0], pb, ps).wait()

    def wait_o(ob, osm):
        pltpu.make_async_copy(ob, out_hbm.at[0], osm).wait()

    def compute(i, gb, ob, pb):
        vvec = valbuf[pl.ds(i * DEG, LANES)]
        vbs = [jnp.broadcast_to(vvec[j], (LANES,)) for j in range(DEG)]

        def fc_fn(t, carry2):
            for u in range(3):
                sl = pl.ds((t * 3 + u) * LANES, LANES)
                acc = vbs[0] * gb[0, sl]
                for j in range(1, DEG):
                    acc = acc + vbs[j] * gb[j, sl]
                if minus:
                    acc = 2.0 * acc - pb[sl]
                ob[sl] = acc
            return carry2

        lax.fori_loop(0, FCH // 3, fc_fn, 0)

    start_g(0, gbufA, gsemA)
    start_g(1, gbufB, gsemB)
    if minus:
        start_p(0, pbufA, psemA)
        start_p(1, pbufB, psemB)

    npairs = _RPW // 2

    def pair_fn(p, carry):
        slots = ((gbufA, gsemA, obufA, osemA, pbufA, psemA),
                 (gbufB, gsemB, obufB, osemB, pbufB, psemB))
        for u, (gb, gs, ob, osm, pb, ps) in enumerate(slots):
            i = p * 2 + u
            wait_g(gb, gs)
            if minus:
                wait_p(pb, ps)

            @pl.when(p > 0)
            def _():
                wait_o(ob, osm)

            compute(i, gb, ob, pb)
            pltpu.async_copy(ob, out_hbm.at[base_row + i], osm)

            @pl.when(p < npairs - 1)
            def _():
                start_g(i + 2, gb, gs)
                if minus:
                    start_p(i + 2, pb, ps)
        return carry

    lax.fori_loop(0, npairs, pair_fn, 0)
    wait_o(obufA, osemA)
    wait_o(obufB, osemB)


_SLAB_OFF = (0, 144, 272, 400)
_SLAB_W = (144, 128, 128, 128)
_SLAB_MAX = 144
_SRC_PER_TILE = N // 16
_XCH = 25
_NCHUNK = _SRC_PER_TILE // _XCH


def _a2_slab(off, w, minus, s, x_hbm, vals_hbm_ref, xp_hbm, out_hbm, shared,
             colbuf, valbuf, xcbuf, sbuf, wbuf, pbuf, ssems):
    nq = w // LANES
    tbase = s * _SRC_PER_TILE

    def zb_fn(i, carry):
        zv = jnp.zeros((LANES,), jnp.float32)
        for q in range(_SLAB_MAX // LANES):
            xcbuf[i, pl.ds(q * LANES, LANES)] = zv
        return carry

    lax.fori_loop(0, _XCH, zb_fn, 0)

    def zc_fn(z, carry):
        pltpu.sync_copy(xcbuf, shared.at[pl.ds(tbase + z * _XCH, _XCH)])
        return carry

    lax.fori_loop(0, _NCHUNK, zc_fn, 0)
    plsc.subcore_barrier()

    def wait_s(u):
        pltpu.make_async_copy(sbuf.at[u], shared.at[colbuf.at[0]],
                              ssems[u]).wait()

    def chunk_fn(k, carry):
        r0 = tbase + k * _XCH
        pltpu.sync_copy(x_hbm.at[pl.ds(r0, _XCH), pl.ds(off, w)],
                        xcbuf.at[:, pl.ds(0, w)])
        pltpu.sync_copy(vals_hbm_ref.at[pl.ds(r0, _XCH)], valbuf)

        def grp_fn(g, carry2):
            for u in range(5):
                lr = g * 5 + u

                @pl.when(k + g > 0)
                def _():
                    wait_s(u)

                vv = valbuf[lr]
                vbs = [jnp.broadcast_to(vv[j], (LANES,)) for j in range(DEG)]

                def q_fn(q, carry3):
                    sl = pl.ds(q * LANES, LANES)
                    xq = xcbuf[lr, sl]
                    for j in range(DEG):
                        sbuf[u, j, sl] = vbs[j] * xq
                    return carry3

                lax.fori_loop(0, nq, q_fn, 0)
                idx = colbuf.at[k * _XCH + lr]
                pltpu.async_copy(sbuf.at[u], shared.at[idx], ssems[u],
                                 add=True)
            return carry2

        lax.fori_loop(0, _XCH // 5, grp_fn, 0)
        return carry

    lax.fori_loop(0, _NCHUNK, chunk_fn, 0)
    for u in range(5):
        wait_s(u)
    plsc.subcore_barrier()

    def wchunk_fn(z, carry):
        rows = pl.ds(tbase + z * _XCH, _XCH)
        if minus:
            pltpu.sync_copy(shared.at[rows, pl.ds(0, w)], wbuf.at[:, pl.ds(0, w)])
            pltpu.sync_copy(xp_hbm.at[rows, pl.ds(off, w)], pbuf.at[:, pl.ds(0, w)])

            def wb_fn(i, carry3):
                for q in range(nq):
                    sl = pl.ds(q * LANES, LANES)
                    wbuf[i, sl] = 2.0 * wbuf[i, sl] - pbuf[i, sl]
                return carry3

            lax.fori_loop(0, _XCH, wb_fn, 0)
            pltpu.sync_copy(wbuf.at[:, pl.ds(0, w)], out_hbm.at[rows, pl.ds(off, w)])
        else:
            pltpu.sync_copy(shared.at[rows, pl.ds(0, w)], out_hbm.at[rows, pl.ds(off, w)])
        return carry

    lax.fori_loop(0, _NCHUNK, wchunk_fn, 0)


def _a2_body(minus, *refs):
    if minus:
        (x_hbm, cols_hbm, vals_hbm, xp_hbm, out_hbm,
         shared, colbuf, valbuf, xcbuf, sbuf, wbuf, pbuf,
         sm0, sm1, sm2, sm3, sm4) = refs
    else:
        (x_hbm, cols_hbm, vals_hbm, out_hbm,
         shared, colbuf, valbuf, xcbuf, sbuf, wbuf,
         sm0, sm1, sm2, sm3, sm4) = refs
        xp_hbm = None
        pbuf = None
    ssems = (sm0, sm1, sm2, sm3, sm4)
    c = lax.axis_index("c")
    s = lax.axis_index("s")
    tbase = s * _SRC_PER_TILE
    pltpu.sync_copy(cols_hbm.at[pl.ds(tbase, _SRC_PER_TILE)], colbuf)
    for s_loc in range(2):
        for cc in range(2):
            sid = cc * 2 + s_loc

            @pl.when(c == cc)
            def _():
                _a2_slab(_SLAB_OFF[sid], _SLAB_W[sid], minus, s, x_hbm,
                         vals_hbm, xp_hbm, out_hbm, shared, colbuf, valbuf,
                         xcbuf, sbuf, wbuf, pbuf, ssems)


def _sc_spmm_a2(x, cols2d, vals2d, xprev=None):
    minus = xprev is not None
    scratch = [
        pltpu.VMEM_SHARED((N, _SLAB_MAX), jnp.float32),
        pltpu.VMEM((_SRC_PER_TILE, DEG), jnp.int32),
        pltpu.VMEM((_XCH, DEG), jnp.float32),
        pltpu.VMEM((_XCH, _SLAB_MAX), jnp.float32),
        pltpu.VMEM((5, DEG, _SLAB_MAX), jnp.float32),
        pltpu.VMEM((_XCH, _SLAB_MAX), jnp.float32),
    ]
    if minus:
        scratch.append(pltpu.VMEM((_XCH, _SLAB_MAX), jnp.float32))
    scratch.extend([pltpu.SemaphoreType.DMA] * 5)
    kfn = pl.kernel(
        functools.partial(_a2_body, minus),
        mesh=plsc.VectorSubcoreMesh(core_axis_name="c", subcore_axis_name="s"),
        out_type=jax.ShapeDtypeStruct((N, FW), jnp.float32),
        scratch_types=scratch,
        compiler_params=pltpu.CompilerParams(use_tc_tiling_on_sc=False),
    )
    args = (x, cols2d, vals2d) + ((xprev,) if minus else ())
    return kfn(*args)


def _sc_spmm_a1(x, cols, vals, xprev=None):
    minus = xprev is not None
    scratch = [
        pltpu.VMEM((_ECH,), jnp.int32),
        pltpu.VMEM((_ECH,), jnp.float32),
        pltpu.VMEM((DEG, FW), jnp.float32),
        pltpu.VMEM((DEG, FW), jnp.float32),
        pltpu.VMEM((FW,), jnp.float32),
        pltpu.VMEM((FW,), jnp.float32),
    ]
    if minus:
        scratch.append(pltpu.VMEM((FW,), jnp.float32))
        scratch.append(pltpu.VMEM((FW,), jnp.float32))
    nsem = 6 if minus else 4
    scratch.extend([pltpu.SemaphoreType.DMA] * nsem)
    kfn = pl.kernel(
        functools.partial(_a1_body, minus),
        mesh=plsc.VectorSubcoreMesh(core_axis_name="c", subcore_axis_name="s"),
        out_type=jax.ShapeDtypeStruct((_NP, FW), jnp.float32),
        scratch_types=scratch,
        compiler_params=pltpu.CompilerParams(use_tc_tiling_on_sc=False),
    )
    args = (x, cols, vals) + ((xprev,) if minus else ())
    return kfn(*args)


def _diffuse(x0, cols0p, vals0p, dst2d, vals2d):
    y1 = _sc_spmm_a1(x0, cols0p, vals0p)
    y2 = _sc_spmm_a1(y1, cols0p, vals0p, xprev=x0)
    y3 = _sc_spmm_a2(x0, dst2d, vals2d)
    y4 = _sc_spmm_a2(y3, dst2d, vals2d, xprev=x0)
    return [x0, y1, y2, y3, y4]


def _gate_body(x0_ref, x1_ref, x2_ref, x3_ref, x4_ref, w_ref, bias_ref, st_ref,
               rs_ref, u_ref):
    acc = jnp.broadcast_to(bias_ref[...], (RB, 2 * U)).astype(jnp.float32)
    for m, xr in enumerate((x0_ref, x1_ref, x2_ref, x3_ref, x4_ref)):
        acc = acc + jnp.dot(xr[...], w_ref[m], preferred_element_type=jnp.float32)
    val = jax.nn.sigmoid(acc)
    r = val[:, :U]
    u = val[:, U:]
    rs_ref[...] = r * st_ref[...]
    u_ref[...] = u


def _cand_body(x0_ref, x1_ref, x2_ref, x3_ref, x4_ref, w_ref, bias_ref, u_ref,
               st_ref, ns_ref):
    acc = jnp.broadcast_to(bias_ref[...], (RB, U)).astype(jnp.float32)
    for m, xr in enumerate((x0_ref, x1_ref, x2_ref, x3_ref, x4_ref)):
        acc = acc + jnp.dot(xr[...], w_ref[m], preferred_element_type=jnp.float32)
    c = jnp.tanh(acc)
    u = u_ref[...]
    ns_ref[...] = u * st_ref[...] + (1.0 - u) * c


def _row_spec():
    return pl.BlockSpec((RB, ISZ), lambda i: (i, 0))


def _vec_spec(width):
    return pl.BlockSpec((RB, width), lambda i: (i, 0))


def _full_spec(shape):
    return pl.BlockSpec(shape, lambda i: (0,) * len(shape))


def _gate_call(xs_nb, w_g, b_g, st_nb):
    grid = (N * B // RB,)
    return pl.pallas_call(
        _gate_body,
        grid=grid,
        in_specs=[_row_spec()] * 5 + [_full_spec((M, ISZ, 2 * U)), _full_spec((1, 2 * U)), _vec_spec(U)],
        out_specs=[_vec_spec(U), _vec_spec(U)],
        out_shape=[jax.ShapeDtypeStruct((N * B, U), jnp.float32),
                   jax.ShapeDtypeStruct((N * B, U), jnp.float32)],
    )(*xs_nb, w_g, b_g, st_nb)


def _cand_call(xs_nb, w_c, b_c, u_nb, st_nb):
    grid = (N * B // RB,)
    return pl.pallas_call(
        _cand_body,
        grid=grid,
        in_specs=[_row_spec()] * 5 + [_full_spec((M, ISZ, U)), _full_spec((1, U)), _vec_spec(U), _vec_spec(U)],
        out_specs=_vec_spec(U),
        out_shape=jax.ShapeDtypeStruct((N * B, U), jnp.float32),
    )(*xs_nb, w_c, b_c, u_nb, st_nb)


def kernel(inputs, state, sup_rows, sup_cols, sup_vals, W_gate, b_gate, W_cand, b_cand):
    inp_nbf = jnp.transpose(inputs.reshape(B, N, IN_DIM), (1, 0, 2))
    st_nbf = jnp.transpose(state.reshape(B, N, U), (1, 0, 2))
    x0 = jnp.concatenate([inp_nbf, st_nbf], axis=2).reshape(N, B * ISZ)

    wg = W_gate.reshape(ISZ, M, 2 * U).transpose(1, 0, 2)
    wc = W_cand.reshape(ISZ, M, U).transpose(1, 0, 2)

    cols0p = jnp.pad(sup_cols[0], (0, _EPAD - NNZ))
    vals0p = jnp.pad(sup_vals[0], (0, _EPAD - NNZ))
    dst2d = sup_rows[1].reshape(NNZ // DEG, DEG)
    vals2d = sup_vals[1].reshape(NNZ // DEG, DEG)

    xs = _diffuse(x0, cols0p, vals0p, dst2d, vals2d)
    xs_nb = [x.reshape(-1, ISZ) for x in xs]
    st_nb = st_nbf.reshape(N * B, U)

    rs_nb, u_nb = _gate_call(xs_nb, wg, b_gate.reshape(1, 2 * U), st_nb)

    x0c = jnp.concatenate([inp_nbf.reshape(N * B, IN_DIM), rs_nb], axis=1).reshape(N, B * ISZ)
    xs2 = _diffuse(x0c, cols0p, vals0p, dst2d, vals2d)
    xs2_nb = [x.reshape(-1, ISZ) for x in xs2]

    ns_nb = _cand_call(xs2_nb, wc, b_cand.reshape(1, U), u_nb, st_nb)
    ns = jnp.transpose(ns_nb.reshape(N, B, U), (1, 0, 2)).reshape(B, N * U)
    return (ns, ns)


# speedup vs baseline: 4.4499x; 1.0712x over previous
import functools

import jax
import jax.numpy as jnp
from jax import lax
from jax.experimental import pallas as pl
from jax.experimental.pallas import tpu as pltpu
from jax.experimental.pallas import tpu_sc as plsc

N = 10000
DEG = 16
NNZ = N * DEG
B = 8
IN_DIM = 2
U = 64
K = 2
NS = 2
ISZ = IN_DIM + U
M = NS * K + 1
RB = 2000


def _spmm(r, c, v, x):
    return jax.ops.segment_sum(v[:, None] * jnp.take(x, c, axis=0), r, num_segments=N)


LANES = 16
FW = B * ISZ
FCH = FW // LANES
_NW = 32
_NP = 10112
_RPW = _NP // _NW
_ECH = _RPW * DEG
_EPAD = _NP * DEG


def _a1_body(minus, *refs):
    if minus:
        (x_hbm, cols_hbm, vals_hbm, xp_hbm, out_hbm, colbuf, valbuf,
         gbufA, gbufB, obufA, obufB, pbufA, pbufB,
         gsemA, gsemB, osemA, osemB, psemA, psemB) = refs
    else:
        (x_hbm, cols_hbm, vals_hbm, out_hbm, colbuf, valbuf,
         gbufA, gbufB, obufA, obufB,
         gsemA, gsemB, osemA, osemB) = refs
        xp_hbm = pbufA = pbufB = psemA = psemB = None
    c = lax.axis_index("c")
    s = lax.axis_index("s")
    wid = s * 2 + c
    base_row = wid * _RPW
    ebase = base_row * DEG
    pltpu.sync_copy(cols_hbm.at[pl.ds(ebase, _ECH)], colbuf)
    pltpu.sync_copy(vals_hbm.at[pl.ds(ebase, _ECH)], valbuf)

    def start_g(i, gb, gs):
        pltpu.async_copy(x_hbm.at[colbuf.at[pl.ds(i * DEG, DEG)]], gb, gs)

    def start_p(i, pb, ps):
        rowc = jnp.minimum(base_row + i, N - 1)
        pltpu.async_copy(xp_hbm.at[rowc], pb, ps)

    def wait_g(gb, gs):
        pltpu.make_async_copy(x_hbm.at[colbuf.at[pl.ds(0, DEG)]], gb, gs).wait()

    def wait_p(pb, ps):
        pltpu.make_async_copy(xp_hbm.at[0], pb, ps).wait()

    def wait_o(ob, osm):
        pltpu.make_async_copy(ob, out_hbm.at[0], osm).wait()

    def compute(i, gb, ob, pb):
        vvec = valbuf[pl.ds(i * DEG, LANES)]
        vbs = [jnp.broadcast_to(vvec[j], (LANES,)) for j in range(DEG)]

        def fc_fn(t, carry2):
            for u in range(3):
                sl = pl.ds((t * 3 + u) * LANES, LANES)
                acc = vbs[0] * gb[0, sl]
                for j in range(1, DEG):
                    acc = acc + vbs[j] * gb[j, sl]
                if minus:
                    acc = 2.0 * acc - pb[sl]
                ob[sl] = acc
            return carry2

        lax.fori_loop(0, FCH // 3, fc_fn, 0)

    start_g(0, gbufA, gsemA)
    start_g(1, gbufB, gsemB)
    if minus:
        start_p(0, pbufA, psemA)
        start_p(1, pbufB, psemB)

    npairs = _RPW // 2

    def pair_fn(p, carry):
        slots = ((gbufA, gsemA, obufA, osemA, pbufA, psemA),
                 (gbufB, gsemB, obufB, osemB, pbufB, psemB))
        for u, (gb, gs, ob, osm, pb, ps) in enumerate(slots):
            i = p * 2 + u
            wait_g(gb, gs)
            if minus:
                wait_p(pb, ps)

            @pl.when(p > 0)
            def _():
                wait_o(ob, osm)

            compute(i, gb, ob, pb)
            pltpu.async_copy(ob, out_hbm.at[base_row + i], osm)

            @pl.when(p < npairs - 1)
            def _():
                start_g(i + 2, gb, gs)
                if minus:
                    start_p(i + 2, pb, ps)
        return carry

    lax.fori_loop(0, npairs, pair_fn, 0)
    wait_o(obufA, osemA)
    wait_o(obufB, osemB)


_SLAB_OFF = (0, 144, 272, 400)
_SLAB_W = (144, 128, 128, 128)
_SLAB_MAX = 144
_SRC_PER_TILE = N // 16
_XCH = 25
_NCHUNK = _SRC_PER_TILE // _XCH


def _a2_slab(off, w, minus, s, x_hbm, vals_hbm_ref, xp_hbm, out_hbm, shared,
             colbuf, valbuf, xcbuf, sbuf, wbuf, pbuf, ssems):
    nq = w // LANES
    tbase = s * _SRC_PER_TILE

    def zb_fn(i, carry):
        zv = jnp.zeros((LANES,), jnp.float32)
        for q in range(_SLAB_MAX // LANES):
            xcbuf[i, pl.ds(q * LANES, LANES)] = zv
        return carry

    lax.fori_loop(0, _XCH, zb_fn, 0)

    def zc_fn(z, carry):
        pltpu.sync_copy(xcbuf, shared.at[pl.ds(tbase + z * _XCH, _XCH)])
        return carry

    lax.fori_loop(0, _NCHUNK, zc_fn, 0)
    plsc.subcore_barrier()

    def wait_s(u):
        pltpu.make_async_copy(sbuf.at[u], shared.at[colbuf.at[0]],
                              ssems[u]).wait()

    def chunk_fn(k, carry):
        r0 = tbase + k * _XCH
        pltpu.sync_copy(x_hbm.at[pl.ds(r0, _XCH), pl.ds(off, w)],
                        xcbuf.at[:, pl.ds(0, w)])
        pltpu.sync_copy(vals_hbm_ref.at[pl.ds(r0, _XCH)], valbuf)

        def grp_fn(g, carry2):
            for u in range(5):
                lr = g * 5 + u

                @pl.when(k + g > 0)
                def _():
                    wait_s(u)

                vv = valbuf[lr]
                vbs = [jnp.broadcast_to(vv[j], (LANES,)) for j in range(DEG)]

                def q_fn(q, carry3):
                    sl = pl.ds(q * LANES, LANES)
                    xq = xcbuf[lr, sl]
                    for j in range(DEG):
                        sbuf[u, j, sl] = vbs[j] * xq
                    return carry3

                lax.fori_loop(0, nq, q_fn, 0)
                idx = colbuf.at[k * _XCH + lr]
                pltpu.async_copy(sbuf.at[u], shared.at[idx], ssems[u],
                                 add=True)
            return carry2

        lax.fori_loop(0, _XCH // 5, grp_fn, 0)
        return carry

    lax.fori_loop(0, _NCHUNK, chunk_fn, 0)
    for u in range(5):
        wait_s(u)
    plsc.subcore_barrier()

    def wchunk_fn(z, carry):
        rows = pl.ds(tbase + z * _XCH, _XCH)
        if minus:
            pltpu.sync_copy(shared.at[rows, pl.ds(0, w)], wbuf.at[:, pl.ds(0, w)])
            pltpu.sync_copy(xp_hbm.at[rows, pl.ds(off, w)], pbuf.at[:, pl.ds(0, w)])

            def wb_fn(i, carry3):
                for q in range(nq):
                    sl = pl.ds(q * LANES, LANES)
                    wbuf[i, sl] = 2.0 * wbuf[i, sl] - pbuf[i, sl]
                return carry3

            lax.fori_loop(0, _XCH, wb_fn, 0)
            pltpu.sync_copy(wbuf.at[:, pl.ds(0, w)], out_hbm.at[rows, pl.ds(off, w)])
        else:
            pltpu.sync_copy(shared.at[rows, pl.ds(0, w)], out_hbm.at[rows, pl.ds(off, w)])
        return carry

    lax.fori_loop(0, _NCHUNK, wchunk_fn, 0)


def _a2_body(minus, *refs):
    if minus:
        (x_hbm, cols_hbm, vals_hbm, xp_hbm, out_hbm,
         shared, colbuf, valbuf, xcbuf, sbuf, wbuf, pbuf,
         sm0, sm1, sm2, sm3, sm4) = refs
    else:
        (x_hbm, cols_hbm, vals_hbm, out_hbm,
         shared, colbuf, valbuf, xcbuf, sbuf, wbuf,
         sm0, sm1, sm2, sm3, sm4) = refs
        xp_hbm = None
        pbuf = None
    ssems = (sm0, sm1, sm2, sm3, sm4)
    c = lax.axis_index("c")
    s = lax.axis_index("s")
    tbase = s * _SRC_PER_TILE
    pltpu.sync_copy(cols_hbm.at[pl.ds(tbase, _SRC_PER_TILE)], colbuf)
    for s_loc in range(2):
        for cc in range(2):
            sid = cc * 2 + s_loc

            @pl.when(c == cc)
            def _():
                _a2_slab(_SLAB_OFF[sid], _SLAB_W[sid], minus, s, x_hbm,
                         vals_hbm, xp_hbm, out_hbm, shared, colbuf, valbuf,
                         xcbuf, sbuf, wbuf, pbuf, ssems)


def _sc_spmm_a2(x, cols2d, vals2d, xprev=None):
    minus = xprev is not None
    scratch = [
        pltpu.VMEM_SHARED((N, _SLAB_MAX), jnp.float32),
        pltpu.VMEM((_SRC_PER_TILE, DEG), jnp.int32),
        pltpu.VMEM((_XCH, DEG), jnp.float32),
        pltpu.VMEM((_XCH, _SLAB_MAX), jnp.float32),
        pltpu.VMEM((5, DEG, _SLAB_MAX), jnp.float32),
        pltpu.VMEM((_XCH, _SLAB_MAX), jnp.float32),
    ]
    if minus:
        scratch.append(pltpu.VMEM((_XCH, _SLAB_MAX), jnp.float32))
    scratch.extend([pltpu.SemaphoreType.DMA] * 5)
    kfn = pl.kernel(
        functools.partial(_a2_body, minus),
        mesh=plsc.VectorSubcoreMesh(core_axis_name="c", subcore_axis_name="s"),
        out_type=jax.ShapeDtypeStruct((N, FW), jnp.float32),
        scratch_types=scratch,
        compiler_params=pltpu.CompilerParams(use_tc_tiling_on_sc=False),
    )
    args = (x, cols2d, vals2d) + ((xprev,) if minus else ())
    return kfn(*args)


def _sc_spmm_a1(x, cols, vals, xprev=None):
    minus = xprev is not None
    scratch = [
        pltpu.VMEM((_ECH,), jnp.int32),
        pltpu.VMEM((_ECH,), jnp.float32),
        pltpu.VMEM((DEG, FW), jnp.float32),
        pltpu.VMEM((DEG, FW), jnp.float32),
        pltpu.VMEM((FW,), jnp.float32),
        pltpu.VMEM((FW,), jnp.float32),
    ]
    if minus:
        scratch.append(pltpu.VMEM((FW,), jnp.float32))
        scratch.append(pltpu.VMEM((FW,), jnp.float32))
    nsem = 6 if minus else 4
    scratch.extend([pltpu.SemaphoreType.DMA] * nsem)
    kfn = pl.kernel(
        functools.partial(_a1_body, minus),
        mesh=plsc.VectorSubcoreMesh(core_axis_name="c", subcore_axis_name="s"),
        out_type=jax.ShapeDtypeStruct((_NP, FW), jnp.float32),
        scratch_types=scratch,
        compiler_params=pltpu.CompilerParams(use_tc_tiling_on_sc=False),
    )
    args = (x, cols, vals) + ((xprev,) if minus else ())
    return kfn(*args)


def _diffuse(x0, cols0p, vals0p, dst2d, vals2d):
    y1 = _sc_spmm_a1(x0, cols0p, vals0p)
    y2 = _sc_spmm_a1(y1, cols0p, vals0p, xprev=x0)
    y3 = _sc_spmm_a2(x0, dst2d, vals2d)
    y4 = _sc_spmm_a2(y3, dst2d, vals2d, xprev=x0)
    return [x0, y1, y2, y3, y4]


PNB = 1000


def _pack_body(inp_ref, st_ref, x0_ref):
    for b in range(B):
        x0_ref[:, pl.ds(b * ISZ, IN_DIM)] = inp_ref[b]
        x0_ref[:, pl.ds(b * ISZ + IN_DIM, U)] = st_ref[b]


def _pack_call(inp3, st3):
    return pl.pallas_call(
        _pack_body,
        grid=(N // PNB,),
        in_specs=[pl.BlockSpec((B, PNB, IN_DIM), lambda i: (0, i, 0)),
                  pl.BlockSpec((B, PNB, U), lambda i: (0, i, 0))],
        out_specs=pl.BlockSpec((PNB, FW), lambda i: (i, 0)),
        out_shape=jax.ShapeDtypeStruct((N, FW), jnp.float32),
    )(inp3, st3)


def _gate_body(x0_ref, x1_ref, x2_ref, x3_ref, x4_ref, w_ref, bias_ref,
               x0c_ref, u_ref):
    acc = jnp.broadcast_to(bias_ref[...], (PNB, B * 2 * U)).astype(jnp.float32)
    for m, xr in enumerate((x0_ref, x1_ref, x2_ref, x3_ref, x4_ref)):
        acc = acc + jnp.dot(xr[...], w_ref[m], preferred_element_type=jnp.float32)
    val = jax.nn.sigmoid(acc)
    x0b = x0_ref[...]
    for b in range(B):
        rb = val[:, b * 2 * U:b * 2 * U + U]
        ub = val[:, b * 2 * U + U:(b + 1) * 2 * U]
        stb = x0b[:, b * ISZ + IN_DIM:(b + 1) * ISZ]
        x0c_ref[:, pl.ds(b * ISZ, IN_DIM)] = x0b[:, b * ISZ:b * ISZ + IN_DIM]
        x0c_ref[:, pl.ds(b * ISZ + IN_DIM, U)] = rb * stb
        u_ref[:, pl.ds(b * U, U)] = ub


def _gate_call(xs, wg_big, bg_big):
    tap_spec = pl.BlockSpec((PNB, FW), lambda i: (i, 0))
    return pl.pallas_call(
        _gate_body,
        grid=(N // PNB,),
        in_specs=[tap_spec] * 5 + [
            pl.BlockSpec((M, FW, B * 2 * U), lambda i: (0, 0, 0)),
            pl.BlockSpec((1, B * 2 * U), lambda i: (0, 0))],
        out_specs=[pl.BlockSpec((PNB, FW), lambda i: (i, 0)),
                   pl.BlockSpec((PNB, B * U), lambda i: (i, 0))],
        out_shape=[jax.ShapeDtypeStruct((N, FW), jnp.float32),
                   jax.ShapeDtypeStruct((N, B * U), jnp.float32)],
    )(*xs, wg_big, bg_big)


def _cand_body(x0_ref, x1_ref, x2_ref, x3_ref, x4_ref, w_ref, bias_ref,
               xorig_ref, u_ref, ns_ref):
    acc = jnp.broadcast_to(bias_ref[...], (PNB, B * U)).astype(jnp.float32)
    for m, xr in enumerate((x0_ref, x1_ref, x2_ref, x3_ref, x4_ref)):
        acc = acc + jnp.dot(xr[...], w_ref[m], preferred_element_type=jnp.float32)
    cc = jnp.tanh(acc)
    x0b = xorig_ref[...]
    ub = u_ref[...]
    for b in range(B):
        stb = x0b[:, b * ISZ + IN_DIM:(b + 1) * ISZ]
        u_b = ub[:, b * U:(b + 1) * U]
        c_b = cc[:, b * U:(b + 1) * U]
        ns_ref[b] = u_b * stb + (1.0 - u_b) * c_b


def _cand_call(xs2, wc_big, bc_big, x0, u_big):
    tap_spec = pl.BlockSpec((PNB, FW), lambda i: (i, 0))
    return pl.pallas_call(
        _cand_body,
        grid=(N // PNB,),
        in_specs=[tap_spec] * 5 + [
            pl.BlockSpec((M, FW, B * U), lambda i: (0, 0, 0)),
            pl.BlockSpec((1, B * U), lambda i: (0, 0)),
            pl.BlockSpec((PNB, FW), lambda i: (i, 0)),
            pl.BlockSpec((PNB, B * U), lambda i: (i, 0))],
        out_specs=pl.BlockSpec((B, PNB, U), lambda i: (0, i, 0)),
        out_shape=jax.ShapeDtypeStruct((B, N, U), jnp.float32),
    )(*xs2, wc_big, bc_big, x0, u_big)


def kernel(inputs, state, sup_rows, sup_cols, sup_vals, W_gate, b_gate, W_cand, b_cand):
    inp3 = inputs.reshape(B, N, IN_DIM)
    st3 = state.reshape(B, N, U)
    x0 = _pack_call(inp3, st3)

    wg = W_gate.reshape(ISZ, M, 2 * U).transpose(1, 0, 2)
    wc = W_cand.reshape(ISZ, M, U).transpose(1, 0, 2)
    eye8 = jnp.eye(B, dtype=jnp.float32)
    wg_big = jnp.einsum('mfo,bd->mbfdo', wg, eye8).reshape(M, FW, B * 2 * U)
    wc_big = jnp.einsum('mfo,bd->mbfdo', wc, eye8).reshape(M, FW, B * U)
    bg_big = jnp.tile(b_gate, B).reshape(1, B * 2 * U)
    bc_big = jnp.tile(b_cand, B).reshape(1, B * U)

    cols0p = jnp.pad(sup_cols[0], (0, _EPAD - NNZ))
    vals0p = jnp.pad(sup_vals[0], (0, _EPAD - NNZ))
    dst2d = sup_rows[1].reshape(NNZ // DEG, DEG)
    vals2d = sup_vals[1].reshape(NNZ // DEG, DEG)

    xs = _diffuse(x0, cols0p, vals0p, dst2d, vals2d)
    x0c, u_big = _gate_call(xs, wg_big, bg_big)
    xs2 = _diffuse(x0c, cols0p, vals0p, dst2d, vals2d)
    ns3 = _cand_call(xs2, wc_big, bc_big, x0, u_big)
    return (ns3.reshape(B, N * U),) * 2
